# baseline (speedup 1.0000x reference)
import functools

import jax
import jax.numpy as jnp
from jax import lax
from jax.experimental import pallas as pl
from jax.experimental.pallas import tpu as pltpu
from jax.experimental.pallas import tpu_sc as plsc

HIDDEN = 2048
L = 16
NC, NS = 2, 16
NW = NC * NS
C = 16
EPS = 1e-12
KCH = HIDDEN // L


def _rsqrt_newton(v):
    vi = plsc.bitcast(v, jnp.int32)
    y = plsc.bitcast(jnp.full((L,), 0x5F3759DF, dtype=jnp.int32) - (vi >> 1),
                     jnp.float32)
    for _ in range(4):
        y = y * (1.5 - 0.5 * v * y * y)
    return y


def _make_kernel(n_tokens, S):
    rows_pw = n_tokens // NW
    n_chunks = rows_pw // C

    @functools.partial(
        pl.kernel,
        out_type=jax.ShapeDtypeStruct((n_tokens, HIDDEN), jnp.float32),
        mesh=plsc.VectorSubcoreMesh(core_axis_name="c", subcore_axis_name="s"),
        compiler_params=pltpu.CompilerParams(needs_layout_passes=False),
        scratch_types=[
            pltpu.VMEM((n_chunks, C), jnp.int32),
            pltpu.VMEM((C, HIDDEN), jnp.float32),
            pltpu.VMEM((C, HIDDEN), jnp.float32),
            pltpu.VMEM((HIDDEN,), jnp.float32),
            pltpu.VMEM((HIDDEN,), jnp.float32),
            pltpu.SemaphoreType.DMA,
            pltpu.SemaphoreType.DMA,
        ],
    )
    def emb(ids_hbm, tok_hbm, pos_hbm, g_hbm, b_hbm, out_hbm,
            ids_v, tok_v, pos_v, g_v, b_v, sem_t, sem_p):
        wid = lax.axis_index("s") * NC + lax.axis_index("c")
        row_base = wid * rows_pw
        pos_base = lax.rem(row_base, S)

        pltpu.sync_copy(ids_hbm.at[wid], ids_v)
        pltpu.sync_copy(g_hbm, g_v)
        pltpu.sync_copy(b_hbm, b_v)

        def chunk(j, _):
            cp_t = pltpu.async_copy(tok_hbm.at[ids_v.at[j]], tok_v, sem_t)
            cp_p = pltpu.async_copy(
                pos_hbm.at[pl.ds(pos_base + j * C, C)], pos_v, sem_p)
            cp_t.wait()
            cp_p.wait()

            def row(r, _):
                def stats(k, carry):
                    acc, acc2 = carry
                    sl = pl.ds(pl.multiple_of(k * L, L), L)
                    x = tok_v[r, sl] + pos_v[r, sl]
                    tok_v[r, sl] = x
                    return acc + x, acc2 + x * x

                zero = jnp.zeros((L,), jnp.float32)
                acc, acc2 = lax.fori_loop(0, KCH, stats, (zero, zero))
                mean = jnp.sum(acc) * (1.0 / HIDDEN)
                var = jnp.sum(acc2) * (1.0 / HIDDEN) - mean * mean
                rstd = _rsqrt_newton(jnp.broadcast_to(var + EPS, (L,)))
                mv = jnp.broadcast_to(mean, (L,))

                def norm(k, _):
                    sl = pl.ds(pl.multiple_of(k * L, L), L)
                    x = tok_v[r, sl]
                    tok_v[r, sl] = (x - mv) * rstd * g_v[sl] + b_v[sl]
                    return 0

                lax.fori_loop(0, KCH, norm, 0)
                return 0

            lax.fori_loop(0, C, row, 0)
            pltpu.sync_copy(tok_v, out_hbm.at[pl.ds(row_base + j * C, C)])
            return 0

        lax.fori_loop(0, n_chunks, chunk, 0)

    return emb


def kernel(input_ids, token_table, pos_table, ln_gamma, ln_beta):
    B, S = input_ids.shape
    n = B * S
    ids = input_ids.reshape(NW, (n // NW) // C, C).astype(jnp.int32)
    out = _make_kernel(n, S)(ids, token_table, pos_table,
                             ln_gamma.astype(jnp.float32),
                             ln_beta.astype(jnp.float32))
    return out.reshape(B, S, HIDDEN)


# speedup vs baseline: 5.2632x; 5.2632x over previous
import functools

import jax
import jax.numpy as jnp
from jax import lax
from jax.experimental import pallas as pl
from jax.experimental.pallas import tpu as pltpu
from jax.experimental.pallas import tpu_sc as plsc

HIDDEN = 2048
NC, NS = 2, 16
NW = NC * NS
C = 16
EPS = 1e-12
SEQ_BLK = 256


def _make_gather(n_tokens):
    rows_pw = n_tokens // NW
    n_chunks = rows_pw // C

    @functools.partial(
        pl.kernel,
        out_type=jax.ShapeDtypeStruct((n_tokens, HIDDEN), jnp.float32),
        mesh=plsc.VectorSubcoreMesh(core_axis_name="c", subcore_axis_name="s"),
        compiler_params=pltpu.CompilerParams(needs_layout_passes=False),
        scratch_types=[
            pltpu.VMEM((n_chunks, C), jnp.int32),
            pltpu.VMEM((C, HIDDEN), jnp.float32),
            pltpu.VMEM((C, HIDDEN), jnp.float32),
            pltpu.SemaphoreType.DMA,
            pltpu.SemaphoreType.DMA,
            pltpu.SemaphoreType.DMA,
            pltpu.SemaphoreType.DMA,
        ],
    )
    def gather(ids_hbm, tok_hbm, out_hbm,
               ids_v, buf0, buf1, sg0, sg1, so0, so1):
        wid = lax.axis_index("s") * NC + lax.axis_index("c")
        row_base = wid * rows_pw
        pltpu.sync_copy(ids_hbm.at[wid], ids_v)

        bufs = (buf0, buf1)
        gsems = (sg0, sg1)
        osems = (so0, so1)

        def start_gather(j, b):
            return pltpu.async_copy(tok_hbm.at[ids_v.at[j]], bufs[b],
                                    gsems[b])

        def start_out(j, b):
            return pltpu.async_copy(
                bufs[b], out_hbm.at[pl.ds(row_base + j * C, C)], osems[b])

        start_gather(0, 0)
        start_gather(1, 1)

        def wait_gather(j, b):
            pltpu.make_async_copy(tok_hbm.at[ids_v.at[j]], bufs[b],
                                  gsems[b]).wait()

        def wait_out(j, b):
            pltpu.make_async_copy(
                bufs[b], out_hbm.at[pl.ds(row_base + j * C, C)],
                osems[b]).wait()

        def body(m, _):
            for b in (0, 1):
                j = 2 * m + b
                wait_gather(j, b)
                start_out(j, b)
                wait_out(j, b)
                start_gather(j + 2, b)
            return 0

        lax.fori_loop(0, n_chunks // 2 - 1, body, 0)

        for b in (0, 1):
            j = n_chunks - 2 + b
            wait_gather(j, b)
            start_out(j, b)
            wait_out(j, b)

    return gather


def _ln_body(x_ref, pos_ref, g_ref, b_ref, o_ref):
    x = x_ref[...] + pos_ref[...][None, :, :]
    mean = jnp.mean(x, axis=-1, keepdims=True)
    xc = x - mean
    var = jnp.mean(xc * xc, axis=-1, keepdims=True)
    o_ref[...] = (xc * lax.rsqrt(var + EPS) * g_ref[...][None, :, :]
                  + b_ref[...][None, :, :])


def _ln(x, pos_table, g, b, B, S):
    grid = (S // SEQ_BLK,)
    return pl.pallas_call(
        _ln_body,
        grid=grid,
        in_specs=[
            pl.BlockSpec((B, SEQ_BLK, HIDDEN), lambda i: (0, i, 0)),
            pl.BlockSpec((SEQ_BLK, HIDDEN), lambda i: (i, 0)),
            pl.BlockSpec((1, HIDDEN), lambda i: (0, 0)),
            pl.BlockSpec((1, HIDDEN), lambda i: (0, 0)),
        ],
        out_specs=pl.BlockSpec((B, SEQ_BLK, HIDDEN), lambda i: (0, i, 0)),
        out_shape=jax.ShapeDtypeStruct((B, S, HIDDEN), jnp.float32),
    )(x, pos_table, g.reshape(1, HIDDEN), b.reshape(1, HIDDEN))


def kernel(input_ids, token_table, pos_table, ln_gamma, ln_beta):
    B, S = input_ids.shape
    n = B * S
    ids = input_ids.reshape(NW, (n // NW) // C, C).astype(jnp.int32)
    gathered = _make_gather(n)(ids, token_table)
    return _ln(gathered.reshape(B, S, HIDDEN), pos_table,
               ln_gamma.astype(jnp.float32), ln_beta.astype(jnp.float32),
               B, S)
